# Initial kernel scaffold; baseline (speedup 1.0000x reference)
import jax
import jax.numpy as jnp
from jax.experimental import pallas as pl


def kernel(x, edge_index, W1, b1, W2, b2, Wl, bl):
    raise NotImplementedError("write your pallas kernel here")


# speedup vs baseline: 10.0422x; 10.0422x over previous
import functools

import jax
import jax.numpy as jnp
from jax import lax
from jax.experimental import pallas as pl
from jax.experimental.pallas import tpu as pltpu
from jax.experimental.pallas import tpu_sc as plsc

N = 10000
NPAD = 10240
F = 128
FH = F // 2
NC, NS = 2, 16
NW = NC * NS
K = 80
RPT = NPAD // NS
ZR = 128

_mesh = plsc.VectorSubcoreMesh(core_axis_name="c", subcore_axis_name="s")


def _prop_body(x_hbm, gidx_hbm, sidx_hbm, zfeat_hbm, out_hbm,
               gidx_v, sidx_v, buf, zbuf, acc_sh, sem):
    nchunk = gidx_hbm.shape[2]
    cid = lax.axis_index("c")
    sid = lax.axis_index("s")
    base = sid * RPT
    pltpu.sync_copy(zfeat_hbm, zbuf)
    for t in range(RPT // ZR):
        pltpu.sync_copy(zbuf, acc_sh.at[pl.ds(base + t * ZR, ZR)])
    pltpu.sync_copy(gidx_hbm.at[cid, sid], gidx_v)
    pltpu.sync_copy(sidx_hbm.at[sid], sidx_v)
    plsc.subcore_barrier()

    def step(j, _):
        pltpu.async_copy(x_hbm.at[gidx_v.at[j]], buf, sem).wait()
        pltpu.sync_copy(buf, acc_sh.at[sidx_v.at[j]], add=True)
        return _

    lax.fori_loop(0, nchunk, step, None)
    plsc.subcore_barrier()
    for t in range(RPT // ZR):
        off = base + t * ZR
        pltpu.sync_copy(acc_sh.at[pl.ds(off, ZR)],
                        out_hbm.at[cid, pl.ds(off, ZR)])


def _make_prop(nchunk):
    return functools.partial(
        pl.kernel,
        out_type=jax.ShapeDtypeStruct((NC, NPAD, FH), jnp.float32),
        mesh=_mesh,
        scratch_types=[
            pltpu.VMEM((nchunk, K), jnp.int32),
            pltpu.VMEM((nchunk, K), jnp.int32),
            pltpu.VMEM((K, FH), jnp.float32),
            pltpu.VMEM((ZR, FH), jnp.float32),
            pltpu.VMEM_SHARED((NPAD, FH), jnp.float32),
            pltpu.SemaphoreType.DMA,
        ],
        compiler_params=pltpu.CompilerParams(use_tc_tiling_on_sc=False),
    )(_prop_body)


def _hist_body(ridx_hbm, cidx_hbm, ones_hbm, zh_hbm, out_hbm,
               ridx_v, cidx_v, ones_v, zh_v, dacc_sh, bacc_sh):
    nchunk = ridx_hbm.shape[1]
    cid = lax.axis_index("c")
    sid = lax.axis_index("s")
    wid = sid * NC + cid
    base = sid * RPT
    pltpu.sync_copy(zh_hbm, zh_v)
    pltpu.sync_copy(zh_v, dacc_sh.at[pl.ds(base, RPT)])
    pltpu.sync_copy(zh_v, bacc_sh.at[pl.ds(base, RPT)])
    pltpu.sync_copy(ones_hbm, ones_v)
    pltpu.sync_copy(ridx_hbm.at[wid], ridx_v)
    pltpu.sync_copy(cidx_hbm.at[wid], cidx_v)
    plsc.subcore_barrier()

    def step(j, _):
        pltpu.sync_copy(ones_v, dacc_sh.at[ridx_v.at[j]], add=True)
        pltpu.sync_copy(ones_v, bacc_sh.at[cidx_v.at[j]], add=True)
        return _

    lax.fori_loop(0, nchunk, step, None)
    plsc.subcore_barrier()
    pltpu.sync_copy(dacc_sh.at[pl.ds(base, RPT)],
                    out_hbm.at[cid, 0, pl.ds(base, RPT)])
    pltpu.sync_copy(bacc_sh.at[pl.ds(base, RPT)],
                    out_hbm.at[cid, 1, pl.ds(base, RPT)])


def _make_hist(nchunk):
    return functools.partial(
        pl.kernel,
        out_type=jax.ShapeDtypeStruct((NC, 2, NPAD, 16), jnp.float32),
        mesh=_mesh,
        scratch_types=[
            pltpu.VMEM((nchunk, K), jnp.int32),
            pltpu.VMEM((nchunk, K), jnp.int32),
            pltpu.VMEM((K, 16), jnp.float32),
            pltpu.VMEM((RPT, 16), jnp.float32),
            pltpu.VMEM_SHARED((NPAD, 16), jnp.float32),
            pltpu.VMEM_SHARED((NPAD, 16), jnp.float32),
        ],
        compiler_params=pltpu.CompilerParams(use_tc_tiling_on_sc=False),
    )(_hist_body)


BR = 400


def _mm_body(x_ref, w_ref, o_ref):
    o_ref[...] = jnp.dot(x_ref[...], w_ref[...],
                         preferred_element_type=jnp.float32)


def _tc_matmul(x, w):
    return pl.pallas_call(
        _mm_body,
        grid=(N // BR,),
        in_specs=[
            pl.BlockSpec((BR, F), lambda i: (i, 0)),
            pl.BlockSpec((F, F), lambda i: (0, 0)),
        ],
        out_specs=pl.BlockSpec((BR, F), lambda i: (i, 0)),
        out_shape=jax.ShapeDtypeStruct((N, F), jnp.float32),
    )(x, w)


def _inv_from_hist(h_ref):
    cnt = h_ref[0, 0, :, 0:1] + h_ref[1, 0, :, 0:1]
    return jnp.where(cnt == 0.0, 0.0, 1.0 / cnt)


def _scale_body(p_ref, h_ref, o_ref):
    inv = _inv_from_hist(h_ref)
    o_ref[...] = jnp.concatenate([inv * p_ref[0], inv * p_ref[1]], axis=1)


def _tc_scale(p, h, which):
    return pl.pallas_call(
        _scale_body,
        grid=(N // BR,),
        in_specs=[
            pl.BlockSpec((NC, BR, FH), lambda i: (0, i, 0)),
            pl.BlockSpec((NC, 1, BR, 16), lambda i, w=which: (0, w, i, 0)),
        ],
        out_specs=pl.BlockSpec((BR, F), lambda i: (i, 0)),
        out_shape=jax.ShapeDtypeStruct((N, F), jnp.float32),
    )(p, h)


def _convmm_body(p_ref, h_ref, bi_ref, w_ref, bo_ref, o_ref):
    inv = _inv_from_hist(h_ref)
    pre = jnp.concatenate([inv * p_ref[0], inv * p_ref[1]], axis=1)
    hcol = jax.nn.relu(pre + bi_ref[...])
    o_ref[...] = jnp.dot(hcol, w_ref[...],
                         preferred_element_type=jnp.float32) + bo_ref[...]


def _tc_convmm(p, h, which, b_in, w, b_out):
    return pl.pallas_call(
        _convmm_body,
        grid=(N // BR,),
        in_specs=[
            pl.BlockSpec((NC, BR, FH), lambda i: (0, i, 0)),
            pl.BlockSpec((NC, 1, BR, 16), lambda i, w=which: (0, w, i, 0)),
            pl.BlockSpec((1, F), lambda i: (0, 0)),
            pl.BlockSpec((F, F), lambda i: (0, 0)),
            pl.BlockSpec((1, F), lambda i: (0, 0)),
        ],
        out_specs=pl.BlockSpec((BR, F), lambda i: (i, 0)),
        out_shape=jax.ShapeDtypeStruct((N, F), jnp.float32),
    )(p, h, b_in.reshape(1, F), w, b_out.reshape(1, F))


@jax.jit
def kernel(x, edge_index, W1, b1, W2, b2, Wl, bl):
    E = edge_index.shape[1]
    nck = E // NS // K
    nch = E // NW // K

    row = edge_index[0].astype(jnp.int32)
    col = edge_index[1].astype(jnp.int32)
    core = jnp.arange(NC, dtype=jnp.int32).reshape(NC, 1, 1, 1)
    row_g = 2 * row.reshape(1, NS, nck, K) + core
    col_g = 2 * col.reshape(1, NS, nck, K) + core
    row_s = row.reshape(NS, nck, K)
    col_s = col.reshape(NS, nck, K)
    row32 = row.reshape(NW, nch, K)
    col32 = col.reshape(NW, nch, K)

    zfeat = jnp.zeros((ZR, FH), jnp.float32)
    zhist = jnp.zeros((RPT, 16), jnp.float32)
    ones = jnp.ones((K, 16), jnp.float32)
    zb = jnp.zeros((F,), jnp.float32)

    sc_prop = _make_prop(nck)
    hist = _make_hist(nch)(row32, col32, ones, zhist)

    t1 = _tc_matmul(x, W1).reshape(2 * N, FH)
    p1 = sc_prop(t1, row_g, col_s, zfeat)
    e1 = _tc_scale(p1, hist, 1)
    q1 = sc_prop(e1.reshape(2 * N, FH), col_g, row_s, zfeat)
    t2 = _tc_convmm(q1, hist, 0, b1, W2, zb).reshape(2 * N, FH)

    p2 = sc_prop(t2, row_g, col_s, zfeat)
    e2 = _tc_scale(p2, hist, 1)
    q2 = sc_prop(e2.reshape(2 * N, FH), col_g, row_s, zfeat)
    return _tc_convmm(q2, hist, 0, b2, Wl, bl)


# speedup vs baseline: 16.2337x; 1.6165x over previous
import functools

import jax
import jax.numpy as jnp
from jax import lax
from jax.experimental import pallas as pl
from jax.experimental.pallas import tpu as pltpu
from jax.experimental.pallas import tpu_sc as plsc

N = 10000
NPAD = 10240
F = 128
FH = F // 2
NC, NS = 2, 16
NW = NC * NS
K = 80
RPT = NPAD // NS
ZR = 128

_mesh = plsc.VectorSubcoreMesh(core_axis_name="c", subcore_axis_name="s")


def _prop_body(x_hbm, gidx_hbm, sidx_hbm, zfeat_hbm, out_hbm,
               gidx_v, sidx_v, buf_a, buf_b, zbuf, acc_sh, sem_a, sem_b):
    nchunk = gidx_hbm.shape[2]
    cid = lax.axis_index("c")
    sid = lax.axis_index("s")
    base = sid * RPT
    pltpu.sync_copy(zfeat_hbm, zbuf)
    for t in range(RPT // ZR):
        pltpu.sync_copy(zbuf, acc_sh.at[pl.ds(base + t * ZR, ZR)])
    pltpu.sync_copy(gidx_hbm.at[cid, sid], gidx_v)
    pltpu.sync_copy(sidx_hbm.at[sid], sidx_v)
    plsc.subcore_barrier()

    pltpu.async_copy(x_hbm.at[gidx_v.at[0]], buf_a, sem_a)

    def step(jj, _):
        j = 2 * jj
        pltpu.async_copy(x_hbm.at[gidx_v.at[j + 1]], buf_b, sem_b)
        pltpu.make_async_copy(x_hbm.at[gidx_v.at[j]], buf_a, sem_a).wait()
        pltpu.sync_copy(buf_a, acc_sh.at[sidx_v.at[j]], add=True)

        @pl.when(j + 2 < nchunk)
        def _issue():
            pltpu.async_copy(x_hbm.at[gidx_v.at[j + 2]], buf_a, sem_a)

        pltpu.make_async_copy(x_hbm.at[gidx_v.at[j + 1]], buf_b, sem_b).wait()
        pltpu.sync_copy(buf_b, acc_sh.at[sidx_v.at[j + 1]], add=True)
        return _

    lax.fori_loop(0, nchunk // 2, step, None)
    plsc.subcore_barrier()
    for t in range(RPT // ZR):
        off = base + t * ZR
        pltpu.sync_copy(acc_sh.at[pl.ds(off, ZR)],
                        out_hbm.at[cid, pl.ds(off, ZR)])


def _make_prop(nchunk):
    return functools.partial(
        pl.kernel,
        out_type=jax.ShapeDtypeStruct((NC, NPAD, FH), jnp.float32),
        mesh=_mesh,
        scratch_types=[
            pltpu.VMEM((nchunk, K), jnp.int32),
            pltpu.VMEM((nchunk, K), jnp.int32),
            pltpu.VMEM((K, FH), jnp.float32),
            pltpu.VMEM((K, FH), jnp.float32),
            pltpu.VMEM((ZR, FH), jnp.float32),
            pltpu.VMEM_SHARED((NPAD, FH), jnp.float32),
            pltpu.SemaphoreType.DMA,
            pltpu.SemaphoreType.DMA,
        ],
        compiler_params=pltpu.CompilerParams(use_tc_tiling_on_sc=False),
    )(_prop_body)


def _hist_body(ridx_hbm, cidx_hbm, ones_hbm, zh_hbm, out_hbm,
               ridx_v, cidx_v, ones_v, zh_v, dacc_sh, bacc_sh):
    nchunk = ridx_hbm.shape[1]
    cid = lax.axis_index("c")
    sid = lax.axis_index("s")
    wid = sid * NC + cid
    base = sid * RPT
    pltpu.sync_copy(zh_hbm, zh_v)
    pltpu.sync_copy(zh_v, dacc_sh.at[pl.ds(base, RPT)])
    pltpu.sync_copy(zh_v, bacc_sh.at[pl.ds(base, RPT)])
    pltpu.sync_copy(ones_hbm, ones_v)
    pltpu.sync_copy(ridx_hbm.at[wid], ridx_v)
    pltpu.sync_copy(cidx_hbm.at[wid], cidx_v)
    plsc.subcore_barrier()

    def step(j, _):
        pltpu.sync_copy(ones_v, dacc_sh.at[ridx_v.at[j]], add=True)
        pltpu.sync_copy(ones_v, bacc_sh.at[cidx_v.at[j]], add=True)
        return _

    lax.fori_loop(0, nchunk, step, None)
    plsc.subcore_barrier()
    pltpu.sync_copy(dacc_sh.at[pl.ds(base, RPT)],
                    out_hbm.at[cid, 0, pl.ds(base, RPT)])
    pltpu.sync_copy(bacc_sh.at[pl.ds(base, RPT)],
                    out_hbm.at[cid, 1, pl.ds(base, RPT)])


def _make_hist(nchunk):
    return functools.partial(
        pl.kernel,
        out_type=jax.ShapeDtypeStruct((NC, 2, NPAD, 16), jnp.float32),
        mesh=_mesh,
        scratch_types=[
            pltpu.VMEM((nchunk, K), jnp.int32),
            pltpu.VMEM((nchunk, K), jnp.int32),
            pltpu.VMEM((K, 16), jnp.float32),
            pltpu.VMEM((RPT, 16), jnp.float32),
            pltpu.VMEM_SHARED((NPAD, 16), jnp.float32),
            pltpu.VMEM_SHARED((NPAD, 16), jnp.float32),
        ],
        compiler_params=pltpu.CompilerParams(use_tc_tiling_on_sc=False),
    )(_hist_body)


BR = 400


def _mm_body(x_ref, w_ref, o_ref):
    o_ref[...] = jnp.dot(x_ref[...], w_ref[...],
                         preferred_element_type=jnp.float32)


def _tc_matmul(x, w):
    return pl.pallas_call(
        _mm_body,
        grid=(N // BR,),
        in_specs=[
            pl.BlockSpec((BR, F), lambda i: (i, 0)),
            pl.BlockSpec((F, F), lambda i: (0, 0)),
        ],
        out_specs=pl.BlockSpec((BR, F), lambda i: (i, 0)),
        out_shape=jax.ShapeDtypeStruct((N, F), jnp.float32),
    )(x, w)


def _inv_from_hist(h_ref):
    cnt = h_ref[0, 0, :, 0:1] + h_ref[1, 0, :, 0:1]
    return jnp.where(cnt == 0.0, 0.0, 1.0 / cnt)


def _scale_body(p_ref, h_ref, o_ref):
    inv = _inv_from_hist(h_ref)
    o_ref[...] = jnp.concatenate([inv * p_ref[0], inv * p_ref[1]], axis=1)


def _tc_scale(p, h, which):
    return pl.pallas_call(
        _scale_body,
        grid=(N // BR,),
        in_specs=[
            pl.BlockSpec((NC, BR, FH), lambda i: (0, i, 0)),
            pl.BlockSpec((NC, 1, BR, 16), lambda i, w=which: (0, w, i, 0)),
        ],
        out_specs=pl.BlockSpec((BR, F), lambda i: (i, 0)),
        out_shape=jax.ShapeDtypeStruct((N, F), jnp.float32),
    )(p, h)


def _convmm_body(p_ref, h_ref, bi_ref, w_ref, bo_ref, o_ref):
    inv = _inv_from_hist(h_ref)
    pre = jnp.concatenate([inv * p_ref[0], inv * p_ref[1]], axis=1)
    hcol = jax.nn.relu(pre + bi_ref[...])
    o_ref[...] = jnp.dot(hcol, w_ref[...],
                         preferred_element_type=jnp.float32) + bo_ref[...]


def _tc_convmm(p, h, which, b_in, w, b_out):
    return pl.pallas_call(
        _convmm_body,
        grid=(N // BR,),
        in_specs=[
            pl.BlockSpec((NC, BR, FH), lambda i: (0, i, 0)),
            pl.BlockSpec((NC, 1, BR, 16), lambda i, w=which: (0, w, i, 0)),
            pl.BlockSpec((1, F), lambda i: (0, 0)),
            pl.BlockSpec((F, F), lambda i: (0, 0)),
            pl.BlockSpec((1, F), lambda i: (0, 0)),
        ],
        out_specs=pl.BlockSpec((BR, F), lambda i: (i, 0)),
        out_shape=jax.ShapeDtypeStruct((N, F), jnp.float32),
    )(p, h, b_in.reshape(1, F), w, b_out.reshape(1, F))


@jax.jit
def kernel(x, edge_index, W1, b1, W2, b2, Wl, bl):
    E = edge_index.shape[1]
    nck = E // NS // K
    nch = E // NW // K

    row = edge_index[0].astype(jnp.int32)
    col = edge_index[1].astype(jnp.int32)
    core = jnp.arange(NC, dtype=jnp.int32).reshape(NC, 1, 1, 1)
    row_g = 2 * row.reshape(1, NS, nck, K) + core
    col_g = 2 * col.reshape(1, NS, nck, K) + core
    row_s = row.reshape(NS, nck, K)
    col_s = col.reshape(NS, nck, K)
    row32 = row.reshape(NW, nch, K)
    col32 = col.reshape(NW, nch, K)

    zfeat = jnp.zeros((ZR, FH), jnp.float32)
    zhist = jnp.zeros((RPT, 16), jnp.float32)
    ones = jnp.ones((K, 16), jnp.float32)
    zb = jnp.zeros((F,), jnp.float32)

    sc_prop = _make_prop(nck)
    hist = _make_hist(nch)(row32, col32, ones, zhist)

    t1 = _tc_matmul(x, W1).reshape(2 * N, FH)
    p1 = sc_prop(t1, row_g, col_s, zfeat)
    e1 = _tc_scale(p1, hist, 1)
    q1 = sc_prop(e1.reshape(2 * N, FH), col_g, row_s, zfeat)
    t2 = _tc_convmm(q1, hist, 0, b1, W2, zb).reshape(2 * N, FH)

    p2 = sc_prop(t2, row_g, col_s, zfeat)
    e2 = _tc_scale(p2, hist, 1)
    q2 = sc_prop(e2.reshape(2 * N, FH), col_g, row_s, zfeat)
    return _tc_convmm(q2, hist, 0, b2, Wl, bl)


# speedup vs baseline: 17.1038x; 1.0536x over previous
import functools

import jax
import jax.numpy as jnp
from jax import lax
from jax.experimental import pallas as pl
from jax.experimental.pallas import tpu as pltpu
from jax.experimental.pallas import tpu_sc as plsc

N = 10000
NPAD = 10240
F = 128
FH = F // 2
NC, NS = 2, 16
NW = NC * NS
K = 80
RPT = NPAD // NS
ZR = 128

_mesh = plsc.VectorSubcoreMesh(core_axis_name="c", subcore_axis_name="s")


def _accumulate(x_hbm, gidx_hbm, sidx_hbm, zfeat_hbm,
                gidx_v, sidx_v, buf_a, buf_b, acc_sh, sem_a, sem_b,
                cid, sid, base):
    nchunk = gidx_hbm.shape[2]
    for t in range(RPT // ZR):
        pltpu.sync_copy(zfeat_hbm, acc_sh.at[pl.ds(base + t * ZR, ZR)])
    pltpu.sync_copy(gidx_hbm.at[cid, sid], gidx_v)
    pltpu.sync_copy(sidx_hbm.at[sid], sidx_v)
    plsc.subcore_barrier()

    pltpu.async_copy(x_hbm.at[gidx_v.at[0]], buf_a, sem_a)

    def step(jj, _):
        j = 2 * jj
        pltpu.async_copy(x_hbm.at[gidx_v.at[j + 1]], buf_b, sem_b)
        pltpu.make_async_copy(x_hbm.at[gidx_v.at[j]], buf_a, sem_a).wait()
        pltpu.sync_copy(buf_a, acc_sh.at[sidx_v.at[j]], add=True)

        @pl.when(j + 2 < nchunk)
        def _issue():
            pltpu.async_copy(x_hbm.at[gidx_v.at[j + 2]], buf_a, sem_a)

        pltpu.make_async_copy(x_hbm.at[gidx_v.at[j + 1]], buf_b, sem_b).wait()
        pltpu.sync_copy(buf_b, acc_sh.at[sidx_v.at[j + 1]], add=True)
        return _

    lax.fori_loop(0, nchunk // 2, step, None)
    plsc.subcore_barrier()


def _prop_body(x_hbm, gidx_hbm, sidx_hbm, zfeat_hbm, out_hbm,
               gidx_v, sidx_v, buf_a, buf_b, acc_sh, sem_a, sem_b):
    cid = lax.axis_index("c")
    sid = lax.axis_index("s")
    base = sid * RPT
    _accumulate(x_hbm, gidx_hbm, sidx_hbm, zfeat_hbm,
                gidx_v, sidx_v, buf_a, buf_b, acc_sh, sem_a, sem_b,
                cid, sid, base)
    for t in range(RPT // ZR):
        off = base + t * ZR
        pltpu.sync_copy(acc_sh.at[pl.ds(off, ZR)],
                        out_hbm.at[cid, pl.ds(off, ZR)])


def _prop_scaled_body(x_hbm, gidx_hbm, sidx_hbm, zfeat_hbm, inv_hbm, out_hbm,
                      gidx_v, sidx_v, buf_a, buf_b, inv_v, tbuf, acc_sh,
                      sem_a, sem_b):
    cid = lax.axis_index("c")
    sid = lax.axis_index("s")
    base = sid * RPT
    _accumulate(x_hbm, gidx_hbm, sidx_hbm, zfeat_hbm,
                gidx_v, sidx_v, buf_a, buf_b, acc_sh, sem_a, sem_b,
                cid, sid, base)
    pltpu.sync_copy(inv_hbm.at[pl.ds(base, RPT)], inv_v)
    for t in range(RPT // ZR):
        off = base + t * ZR
        pltpu.sync_copy(acc_sh.at[pl.ds(off, ZR)], tbuf)

        def rowbody(r, _, t=t):
            iv = inv_v[t * ZR + r, :]
            for q in range(FH // 16):
                tbuf[r, pl.ds(16 * q, 16)] = tbuf[r, pl.ds(16 * q, 16)] * iv
            return _

        lax.fori_loop(0, ZR, rowbody, None)
        pltpu.sync_copy(tbuf, out_hbm.at[cid, pl.ds(off, ZR)])


def _prop_scratch(nchunk):
    return [
        pltpu.VMEM((nchunk, K), jnp.int32),
        pltpu.VMEM((nchunk, K), jnp.int32),
        pltpu.VMEM((K, FH), jnp.float32),
        pltpu.VMEM((K, FH), jnp.float32),
    ]


def _make_prop(nchunk):
    return functools.partial(
        pl.kernel,
        out_type=jax.ShapeDtypeStruct((NC, NPAD, FH), jnp.float32),
        mesh=_mesh,
        scratch_types=_prop_scratch(nchunk) + [
            pltpu.VMEM_SHARED((NPAD, FH), jnp.float32),
            pltpu.SemaphoreType.DMA,
            pltpu.SemaphoreType.DMA,
        ],
        compiler_params=pltpu.CompilerParams(use_tc_tiling_on_sc=False),
    )(_prop_body)


def _make_prop_scaled(nchunk):
    return functools.partial(
        pl.kernel,
        out_type=jax.ShapeDtypeStruct((NC, NPAD, FH), jnp.float32),
        mesh=_mesh,
        scratch_types=_prop_scratch(nchunk) + [
            pltpu.VMEM((RPT, 16), jnp.float32),
            pltpu.VMEM((ZR, FH), jnp.float32),
            pltpu.VMEM_SHARED((NPAD, FH), jnp.float32),
            pltpu.SemaphoreType.DMA,
            pltpu.SemaphoreType.DMA,
        ],
        compiler_params=pltpu.CompilerParams(use_tc_tiling_on_sc=False),
    )(_prop_scaled_body)


def _hist_body(ridx_hbm, cidx_hbm, ones_hbm, zh_hbm, out_hbm,
               ridx_v, cidx_v, ones_v, zh_v, dacc_sh, bacc_sh):
    nchunk = ridx_hbm.shape[1]
    cid = lax.axis_index("c")
    sid = lax.axis_index("s")
    wid = sid * NC + cid
    base = sid * RPT
    pltpu.sync_copy(zh_hbm, zh_v)
    pltpu.sync_copy(zh_v, dacc_sh.at[pl.ds(base, RPT)])
    pltpu.sync_copy(zh_v, bacc_sh.at[pl.ds(base, RPT)])
    pltpu.sync_copy(ones_hbm, ones_v)
    pltpu.sync_copy(ridx_hbm.at[wid], ridx_v)
    pltpu.sync_copy(cidx_hbm.at[wid], cidx_v)
    plsc.subcore_barrier()

    def step(j, _):
        pltpu.sync_copy(ones_v, dacc_sh.at[ridx_v.at[j]], add=True)
        pltpu.sync_copy(ones_v, bacc_sh.at[cidx_v.at[j]], add=True)
        return _

    lax.fori_loop(0, nchunk, step, None)
    plsc.subcore_barrier()
    pltpu.sync_copy(dacc_sh.at[pl.ds(base, RPT)],
                    out_hbm.at[cid, 0, pl.ds(base, RPT)])
    pltpu.sync_copy(bacc_sh.at[pl.ds(base, RPT)],
                    out_hbm.at[cid, 1, pl.ds(base, RPT)])


def _make_hist(nchunk):
    return functools.partial(
        pl.kernel,
        out_type=jax.ShapeDtypeStruct((NC, 2, NPAD, 16), jnp.float32),
        mesh=_mesh,
        scratch_types=[
            pltpu.VMEM((nchunk, K), jnp.int32),
            pltpu.VMEM((nchunk, K), jnp.int32),
            pltpu.VMEM((K, 16), jnp.float32),
            pltpu.VMEM((RPT, 16), jnp.float32),
            pltpu.VMEM_SHARED((NPAD, 16), jnp.float32),
            pltpu.VMEM_SHARED((NPAD, 16), jnp.float32),
        ],
        compiler_params=pltpu.CompilerParams(use_tc_tiling_on_sc=False),
    )(_hist_body)


BR = 400


def _mm_body(x_ref, w_ref, o_ref):
    o_ref[...] = jnp.dot(x_ref[...], w_ref[...],
                         preferred_element_type=jnp.float32)


def _tc_matmul(x, w):
    return pl.pallas_call(
        _mm_body,
        grid=(N // BR,),
        in_specs=[
            pl.BlockSpec((BR, F), lambda i: (i, 0)),
            pl.BlockSpec((F, F), lambda i: (0, 0)),
        ],
        out_specs=pl.BlockSpec((BR, F), lambda i: (i, 0)),
        out_shape=jax.ShapeDtypeStruct((N, F), jnp.float32),
    )(x, w)


def _inv_from_hist(h_ref):
    cnt = h_ref[0, 0, :, 0:1] + h_ref[1, 0, :, 0:1]
    return jnp.where(cnt == 0.0, 0.0, 1.0 / cnt)


def _inv16_body(h_ref, o_ref):
    cnt = h_ref[0, 0] + h_ref[1, 0]
    o_ref[...] = jnp.where(cnt == 0.0, 0.0, 1.0 / cnt)


def _tc_inv16(h, which):
    brh = 1024
    return pl.pallas_call(
        _inv16_body,
        grid=(NPAD // brh,),
        in_specs=[
            pl.BlockSpec((NC, 1, brh, 16), lambda i, w=which: (0, w, i, 0)),
        ],
        out_specs=pl.BlockSpec((brh, 16), lambda i: (i, 0)),
        out_shape=jax.ShapeDtypeStruct((NPAD, 16), jnp.float32),
    )(h)


def _convmm_body(p_ref, h_ref, bi_ref, w_ref, bo_ref, o_ref):
    inv = _inv_from_hist(h_ref)
    pre = jnp.concatenate([inv * p_ref[0], inv * p_ref[1]], axis=1)
    hcol = jax.nn.relu(pre + bi_ref[...])
    o_ref[...] = jnp.dot(hcol, w_ref[...],
                         preferred_element_type=jnp.float32) + bo_ref[...]


def _tc_convmm(p, h, which, b_in, w, b_out):
    return pl.pallas_call(
        _convmm_body,
        grid=(N // BR,),
        in_specs=[
            pl.BlockSpec((NC, BR, FH), lambda i: (0, i, 0)),
            pl.BlockSpec((NC, 1, BR, 16), lambda i, w=which: (0, w, i, 0)),
            pl.BlockSpec((1, F), lambda i: (0, 0)),
            pl.BlockSpec((F, F), lambda i: (0, 0)),
            pl.BlockSpec((1, F), lambda i: (0, 0)),
        ],
        out_specs=pl.BlockSpec((BR, F), lambda i: (i, 0)),
        out_shape=jax.ShapeDtypeStruct((N, F), jnp.float32),
    )(p, h, b_in.reshape(1, F), w, b_out.reshape(1, F))


@jax.jit
def kernel(x, edge_index, W1, b1, W2, b2, Wl, bl):
    E = edge_index.shape[1]
    nck = E // NS // K
    nch = E // NW // K

    row = edge_index[0].astype(jnp.int32)
    col = edge_index[1].astype(jnp.int32)
    core = jnp.arange(NC, dtype=jnp.int32).reshape(NC, 1, 1, 1)
    row_g = 2 * row.reshape(1, NS, nck, K) + core
    col_gb = col.reshape(1, NS, nck, K) + NPAD * core
    row_s = row.reshape(NS, nck, K)
    col_s = col.reshape(NS, nck, K)
    row32 = row.reshape(NW, nch, K)
    col32 = col.reshape(NW, nch, K)

    zfeat = jnp.zeros((ZR, FH), jnp.float32)
    zhist = jnp.zeros((RPT, 16), jnp.float32)
    ones = jnp.ones((K, 16), jnp.float32)
    zb = jnp.zeros((F,), jnp.float32)

    sc_prop = _make_prop(nck)
    sc_prop_scaled = _make_prop_scaled(nck)
    hist = _make_hist(nch)(row32, col32, ones, zhist)
    invb = _tc_inv16(hist, 1)

    t1 = _tc_matmul(x, W1).reshape(2 * N, FH)
    e1 = sc_prop_scaled(t1, row_g, col_s, zfeat, invb)
    q1 = sc_prop(e1.reshape(NC * NPAD, FH), col_gb, row_s, zfeat)
    t2 = _tc_convmm(q1, hist, 0, b1, W2, zb).reshape(2 * N, FH)

    e2 = sc_prop_scaled(t2, row_g, col_s, zfeat, invb)
    q2 = sc_prop(e2.reshape(NC * NPAD, FH), col_gb, row_s, zfeat)
    return _tc_convmm(q2, hist, 0, b2, Wl, bl)
